# baseline (speedup 1.0000x reference)
import jax
import jax.numpy as jnp
from jax import lax
from jax.experimental import pallas as pl
from jax.experimental.pallas import tpu as pltpu

_B = 16384
_C = 1000
_BR = 2048
_NB = _B // _BR
_K = _B // 2


def _fused_body(x_ref, t_ref, out_ref, loss_scr):
    i = pl.program_id(0)
    x = x_ref[...]
    t = t_ref[...]
    m = jnp.max(x, axis=1)
    e = jnp.exp(x - m[:, None])
    col = lax.broadcasted_iota(jnp.int32, x.shape, 1)
    masked = jnp.where(col == t[:, None], x, 0.0)
    ones = jnp.ones((_C, 1), jnp.float32)
    dims = (((1,), (0,)), ((), ()))
    s = lax.dot_general(e, ones, dims, preferred_element_type=jnp.float32)[:, 0]
    tgt = lax.dot_general(masked, ones, dims, preferred_element_type=jnp.float32)[:, 0]
    loss_scr[pl.ds(i * _BR, _BR)] = m + jnp.log(s) - tgt

    @pl.when(i == _NB - 1)
    def _select():
        v_all = loss_scr[...]
        bits = lax.bitcast_convert_type(v_all, jnp.int32)
        ukey = lax.bitcast_convert_type(
            jnp.where(bits < 0, ~bits, bits | jnp.int32(-2147483648)), jnp.uint32
        )

        def step(j, p):
            c = p | (jnp.uint32(1) << (jnp.uint32(31) - j.astype(jnp.uint32)))
            cnt = jnp.sum((ukey >= c).astype(jnp.int32))
            return jnp.where(cnt >= _K, c, p)

        p = lax.fori_loop(0, 32, step, jnp.uint32(0))
        pi = lax.bitcast_convert_type(p, jnp.int32)
        vbits = jnp.where(pi < 0, pi & jnp.int32(0x7FFFFFFF), ~pi)
        v = lax.bitcast_convert_type(vbits, jnp.float32)
        sel = ukey > p
        cnt_gt = jnp.sum(sel.astype(jnp.int32))
        ssum = jnp.sum(jnp.where(sel, v_all, 0.0))
        rem = (_K - cnt_gt).astype(jnp.float32)
        out_ref[0, 0] = ssum + jnp.where(cnt_gt == _K, 0.0, rem * v)


@jax.jit
def kernel(input, target):
    out = pl.pallas_call(
        _fused_body,
        grid=(_NB,),
        in_specs=[
            pl.BlockSpec((_BR, _C), lambda i: (i, 0)),
            pl.BlockSpec((_BR,), lambda i: (i,)),
        ],
        out_specs=pl.BlockSpec(memory_space=pltpu.SMEM),
        out_shape=jax.ShapeDtypeStruct((1, 1), jnp.float32),
        scratch_shapes=[pltpu.VMEM((_B,), jnp.float32)],
    )(input, target)
    return out[0, 0]


# speedup vs baseline: 1.1356x; 1.1356x over previous
import jax
import jax.numpy as jnp
from jax import lax
from jax.experimental import pallas as pl
from jax.experimental.pallas import tpu as pltpu

_B = 16384
_C = 1000
_BR = 2048
_NB = _B // _BR
_K = _B // 2


def _fused_body(x_ref, t_ref, out_ref, loss_scr):
    i = pl.program_id(0)
    x = x_ref[...]
    t = t_ref[...]
    m = jnp.max(x, axis=1)
    e = jnp.exp(x - m[:, None]).astype(jnp.bfloat16)
    ones = jnp.ones((_C, 1), jnp.bfloat16)
    dims = (((1,), (0,)), ((), ()))
    s = lax.dot_general(e, ones, dims, preferred_element_type=jnp.float32)[:, 0]
    col = lax.broadcasted_iota(jnp.int32, x.shape, 1)
    tgt = jnp.sum(jnp.where(col == t[:, None], x, 0.0), axis=1)
    loss_scr[pl.ds(i * _BR, _BR)] = m + jnp.log(s) - tgt

    @pl.when(i == _NB - 1)
    def _select():
        v_all = loss_scr[...]
        bits = lax.bitcast_convert_type(v_all, jnp.int32)
        ukey = lax.bitcast_convert_type(
            jnp.where(bits < 0, ~bits, bits | jnp.int32(-2147483648)), jnp.uint32
        )

        def step(j, p):
            c = p | (jnp.uint32(1) << (jnp.uint32(31) - j.astype(jnp.uint32)))
            cnt = jnp.sum((ukey >= c).astype(jnp.int32))
            return jnp.where(cnt >= _K, c, p)

        p = lax.fori_loop(0, 32, step, jnp.uint32(0))
        pi = lax.bitcast_convert_type(p, jnp.int32)
        vbits = jnp.where(pi < 0, pi & jnp.int32(0x7FFFFFFF), ~pi)
        v = lax.bitcast_convert_type(vbits, jnp.float32)
        sel = ukey > p
        cnt_gt = jnp.sum(sel.astype(jnp.int32))
        ssum = jnp.sum(jnp.where(sel, v_all, 0.0))
        rem = (_K - cnt_gt).astype(jnp.float32)
        out_ref[0, 0] = ssum + jnp.where(cnt_gt == _K, 0.0, rem * v)


@jax.jit
def kernel(input, target):
    out = pl.pallas_call(
        _fused_body,
        grid=(_NB,),
        in_specs=[
            pl.BlockSpec((_BR, _C), lambda i: (i, 0)),
            pl.BlockSpec((_BR,), lambda i: (i,)),
        ],
        out_specs=pl.BlockSpec(memory_space=pltpu.SMEM),
        out_shape=jax.ShapeDtypeStruct((1, 1), jnp.float32),
        scratch_shapes=[pltpu.VMEM((_B,), jnp.float32)],
    )(input, target)
    return out[0, 0]
